# baseline (speedup 1.0000x reference)
import dataclasses
import functools

import jax
import jax.numpy as jnp
from jax import lax
from jax.experimental import pallas as pl
from jax.experimental.pallas import tpu as pltpu
from jax.experimental.pallas import tpu_sc as plsc

_EPS = 1e-05
_GROUP = 64
_GJ = _GROUP // 16


def _kblk(d, limit=4096):
    best = 128
    for m in range(128, limit + 1, 128):
        if d % m == 0:
            best = m
    return best


def _pass1_body(x_ref, w_ref, logits_ref, rsum_ref, rsumsq_ref, *, nb):
    k = pl.program_id(0)

    @pl.when(k == 0)
    def _():
        logits_ref[...] = jnp.zeros_like(logits_ref)
        rsum_ref[...] = jnp.zeros_like(rsum_ref)
        rsumsq_ref[...] = jnp.zeros_like(rsumsq_ref)

    xb = x_ref[...]
    xb16 = xb.astype(jnp.bfloat16)
    wb16 = w_ref[...].astype(jnp.bfloat16)
    for i in range(nb):
        logits_ref[i] += jax.lax.dot_general(
            wb16, xb16[i],
            (((1,), (0,)), ((), ())), preferred_element_type=jnp.float32)
    rsum_ref[...] += jnp.sum(xb, axis=1)
    rsumsq_ref[...] += jnp.sum(xb * xb, axis=1)


def _pass3_body(x_ref, scale_ref, shift_ref, out_ref):
    scale = scale_ref[...][:, None, :]
    shift = shift_ref[...][:, None, :]
    out_ref[...] = x_ref[...] * scale + shift


def _rsqrt_newton(a):
    i = plsc.bitcast(a, jnp.int32)
    i = jnp.int32(0x5F3759DF) - lax.shift_right_arithmetic(i, 1)
    y = plsc.bitcast(i, jnp.float32)
    half = 0.5 * a
    for _ in range(3):
        y = y * (1.5 - half * y * y)
    return y


def _sc_stats_body(logits_hbm, b_hbm, rsum_hbm, rsumsq_hbm, wrow_hbm,
                   brow_hbm, scale_hbm, shift_hbm,
                   lbuf, rbuf, qbuf, wbuf, bbuf, bvec_v, stage, shared,
                   allbuf, sbuf, meanbuf, invbuf, outs, outsh,
                   *, nb, nc, d):
    cid = lax.axis_index("c")
    sid = lax.axis_index("s")
    cw = nc // 16
    c0 = sid * cw
    iota = lax.iota(jnp.int32, 16)

    pltpu.sync_copy(logits_hbm, lbuf)
    pltpu.sync_copy(rsum_hbm, rbuf)
    pltpu.sync_copy(rsumsq_hbm, qbuf)
    pltpu.sync_copy(wrow_hbm, wbuf)
    pltpu.sync_copy(brow_hbm, bbuf)
    pltpu.sync_copy(b_hbm, bvec_v)

    bregs = [bvec_v[pl.ds(16 * j, 16)] for j in range(_GJ)]
    gvecs = [iota + 16 * j for j in range(_GJ)]
    lane0 = iota == 0
    zero16 = jnp.zeros((16,), jnp.int32)
    for j in range(4):
        sbuf[pl.ds(16 * j, 16)] = zero16

    cnt = [jnp.zeros((16,), jnp.float32) for _ in range(_GJ)]
    gsum = [jnp.zeros((16,), jnp.float32) for _ in range(_GJ)]
    gsumsq = [jnp.zeros((16,), jnp.float32) for _ in range(_GJ)]

    for n in range(nb):
        rv0 = plsc.load_gather(rbuf, [iota + (n * nc + c0)])
        rv1 = plsc.load_gather(rbuf, [iota + (n * nc + c0 + 8)])
        qv0 = plsc.load_gather(qbuf, [iota + (n * nc + c0)])
        qv1 = plsc.load_gather(qbuf, [iota + (n * nc + c0 + 8)])
        for i in range(cw):
            base = n * (_GROUP * nc) + c0 + i
            vals = [
                plsc.load_gather(lbuf, [gvecs[j] * nc + base]) + bregs[j]
                for j in range(_GJ)
            ]
            m = jnp.max(vals[0])
            for j in range(1, _GJ):
                m = jnp.maximum(m, jnp.max(vals[j]))
            s = jnp.full((16,), _GROUP, jnp.int32)
            for j in range(_GJ - 1, -1, -1):
                mask = vals[j] == m
                hit = plsc.all_reduce_population_count(mask) > 0
                idx = plsc.all_reduce_ffs(mask) + 16 * j
                s = jnp.where(hit, idx, s)
            rv = rv0[i] if i < 16 else rv1[i - 8]
            qv = qv0[i] if i < 16 else qv1[i - 8]
            for j in range(_GJ):
                eq = gvecs[j] == s
                cnt[j] = cnt[j] + jnp.where(eq, 1.0, 0.0)
                gsum[j] = gsum[j] + jnp.where(eq, rv, 0.0)
                gsumsq[j] = gsumsq[j] + jnp.where(eq, qv, 0.0)
            plsc.store_scatter(sbuf, [jnp.full((16,), n * cw + i, jnp.int32)],
                               s, mask=lane0)

    for j in range(_GJ):
        stage[pl.ds(16 * j, 16)] = cnt[j]
        stage[pl.ds(64 + 16 * j, 16)] = gsum[j]
        stage[pl.ds(128 + 16 * j, 16)] = gsumsq[j]
    pltpu.sync_copy(stage, shared.at[sid])
    plsc.subcore_barrier()
    pltpu.sync_copy(shared, allbuf)

    for j in range(_GJ):
        cnt_g = jnp.zeros((16,), jnp.float32)
        sum_g = jnp.zeros((16,), jnp.float32)
        sumsq_g = jnp.zeros((16,), jnp.float32)
        for t in range(16):
            cnt_g = cnt_g + allbuf[t, pl.ds(16 * j, 16)]
            sum_g = sum_g + allbuf[t, pl.ds(64 + 16 * j, 16)]
            sumsq_g = sumsq_g + allbuf[t, pl.ds(128 + 16 * j, 16)]
        n_el = cnt_g * d
        mean_g = sum_g / jnp.maximum(n_el, 1.0)
        var_g = (sumsq_g - n_el * mean_g * mean_g) / jnp.maximum(
            n_el - 1.0, 1.0)
        inv_g = _rsqrt_newton(var_g + _EPS)
        meanbuf[pl.ds(16 * j, 16)] = mean_g
        invbuf[pl.ds(16 * j, 16)] = inv_g

    hi = nb * nc - 1
    for k in range(2):
        svec = plsc.load_gather(sbuf, [iota + (cid * cw + 16 * k)])
        mvals = plsc.load_gather(meanbuf, [svec])
        ivals = plsc.load_gather(invbuf, [svec])
        widx = jnp.minimum(iota + (cid * nc + c0 + 16 * k), hi)
        wvals = plsc.load_gather(wbuf, [widx])
        bvals = plsc.load_gather(bbuf, [widx])
        sc = wvals * ivals
        outs[pl.ds(16 * k, 16)] = sc
        outsh[pl.ds(16 * k, 16)] = bvals - mvals * sc
    off = pl.multiple_of((cid * 16 + sid) * 32, 32)
    pltpu.sync_copy(outs, scale_hbm.at[pl.ds(off, 32)])
    pltpu.sync_copy(outsh, shift_hbm.at[pl.ds(off, 32)])


def kernel(x, fc_w, fc_b, weight, bias):
    n, c, h, w = x.shape
    d = h * w
    xm = x.transpose(0, 2, 3, 1).reshape(n, d, c)
    wrow = jnp.broadcast_to(weight.reshape(1, c), (n, c))
    brow = jnp.broadcast_to(bias.reshape(1, c), (n, c))

    kblk = _kblk(d)
    ksteps = d // kblk

    logits, rsum, rsumsq = pl.pallas_call(
        functools.partial(_pass1_body, nb=n),
        grid=(ksteps,),
        in_specs=[
            pl.BlockSpec((n, kblk, c), lambda k: (0, k, 0)),
            pl.BlockSpec((_GROUP, kblk), lambda k: (0, k)),
        ],
        out_specs=[
            pl.BlockSpec((n, _GROUP, c), lambda k: (0, 0, 0)),
            pl.BlockSpec((n, c), lambda k: (0, 0)),
            pl.BlockSpec((n, c), lambda k: (0, 0)),
        ],
        out_shape=[
            jax.ShapeDtypeStruct((n, _GROUP, c), jnp.float32),
            jax.ShapeDtypeStruct((n, c), jnp.float32),
            jax.ShapeDtypeStruct((n, c), jnp.float32),
        ],
        compiler_params=pltpu.CompilerParams(
            dimension_semantics=("arbitrary",),
            vmem_limit_bytes=60 * 1024 * 1024),
    )(xm, fc_w)

    mesh = plsc.VectorSubcoreMesh(core_axis_name="c", subcore_axis_name="s")
    cp = pltpu.CompilerParams()
    if "needs_layout_passes" in pltpu.CompilerParams.__dataclass_fields__:
        cp = dataclasses.replace(cp, needs_layout_passes=False)
    cw = c // 16
    sc_stats = pl.kernel(
        functools.partial(_sc_stats_body, nb=n, nc=c, d=float(d)),
        out_type=[
            jax.ShapeDtypeStruct((n * 16 * 32,), jnp.float32),
            jax.ShapeDtypeStruct((n * 16 * 32,), jnp.float32),
        ],
        mesh=mesh,
        scratch_types=[
            pltpu.VMEM((n * _GROUP * c,), jnp.float32),
            pltpu.VMEM((n * c,), jnp.float32),
            pltpu.VMEM((n * c,), jnp.float32),
            pltpu.VMEM((n * c,), jnp.float32),
            pltpu.VMEM((n * c,), jnp.float32),
            pltpu.VMEM((_GROUP,), jnp.float32),
            pltpu.VMEM((256,), jnp.float32),
            pltpu.VMEM_SHARED((16, 256), jnp.float32),
            pltpu.VMEM((16, 256), jnp.float32),
            pltpu.VMEM((64,), jnp.int32),
            pltpu.VMEM((_GROUP,), jnp.float32),
            pltpu.VMEM((_GROUP,), jnp.float32),
            pltpu.VMEM((32,), jnp.float32),
            pltpu.VMEM((32,), jnp.float32),
        ],
        compiler_params=cp,
    )
    scale, shift = sc_stats(logits.reshape(-1), fc_b,
                            rsum.reshape(-1), rsumsq.reshape(-1),
                            wrow.reshape(-1), brow.reshape(-1))
    scale = scale.reshape(n, 16, 32)[:, :, :cw].reshape(n, c)
    shift = shift.reshape(n, 16, 32)[:, :, :cw].reshape(n, c)

    out = pl.pallas_call(
        _pass3_body,
        grid=(ksteps,),
        in_specs=[
            pl.BlockSpec((n, kblk, c), lambda k: (0, k, 0)),
            pl.BlockSpec((n, c), lambda k: (0, 0)),
            pl.BlockSpec((n, c), lambda k: (0, 0)),
        ],
        out_specs=pl.BlockSpec((n, kblk, c), lambda k: (0, k, 0)),
        out_shape=jax.ShapeDtypeStruct((n, d, c), jnp.float32),
        compiler_params=pltpu.CompilerParams(
            dimension_semantics=("arbitrary",),
            vmem_limit_bytes=60 * 1024 * 1024),
    )(xm, scale, shift)

    return out.reshape(n, h, w, c).transpose(0, 3, 1, 2)


# speedup vs baseline: 1.0363x; 1.0363x over previous
import dataclasses
import functools

import jax
import jax.numpy as jnp
from jax import lax
from jax.experimental import pallas as pl
from jax.experimental.pallas import tpu as pltpu
from jax.experimental.pallas import tpu_sc as plsc

_EPS = 1e-05
_GROUP = 64
_GJ = _GROUP // 16


def _kblk(d, limit=4096):
    best = 128
    for m in range(128, limit + 1, 128):
        if d % m == 0:
            best = m
    return best


def _pass1_body(x_ref, w_ref, logits_ref, rsum_ref, rsumsq_ref, *, nb):
    k = pl.program_id(0)

    @pl.when(k == 0)
    def _():
        logits_ref[...] = jnp.zeros_like(logits_ref)
        rsum_ref[...] = jnp.zeros_like(rsum_ref)
        rsumsq_ref[...] = jnp.zeros_like(rsumsq_ref)

    xb = x_ref[...]
    xb16 = xb.astype(jnp.bfloat16)
    wb16 = w_ref[...].astype(jnp.bfloat16)
    for i in range(nb):
        logits_ref[i] += jax.lax.dot_general(
            wb16, xb16[i],
            (((1,), (0,)), ((), ())), preferred_element_type=jnp.float32)
    rsum_ref[...] += jnp.sum(xb, axis=1)
    rsumsq_ref[...] += jnp.sum(xb * xb, axis=1)


def _pass3_body(x_ref, scale_ref, shift_ref, out_ref):
    scale = scale_ref[...][:, None, :]
    shift = shift_ref[...][:, None, :]
    out_ref[...] = x_ref[...] * scale + shift


def _rsqrt_newton(a):
    i = plsc.bitcast(a, jnp.int32)
    i = jnp.int32(0x5F3759DF) - lax.shift_right_arithmetic(i, 1)
    y = plsc.bitcast(i, jnp.float32)
    half = 0.5 * a
    for _ in range(3):
        y = y * (1.5 - half * y * y)
    return y


def _sc_stats_body(logits_hbm, b_hbm, rsum_hbm, rsumsq_hbm, wrow_hbm,
                   brow_hbm, scale_hbm, shift_hbm,
                   lbuf, rbuf, qbuf, wbuf, bbuf, bvec_v, stage, shared,
                   allbuf, sbuf, meanbuf, invbuf, outs, outsh,
                   *, nb, nc, d):
    cid = lax.axis_index("c")
    sid = lax.axis_index("s")
    cw = nc // 16
    c0 = sid * cw
    iota = lax.iota(jnp.int32, 16)

    for n in range(nb):
        loff = pl.multiple_of(n * (nc * _GROUP) + c0 * _GROUP, 8)
        pltpu.sync_copy(logits_hbm.at[pl.ds(loff, cw * _GROUP)],
                        lbuf.at[pl.ds(n * cw * _GROUP, cw * _GROUP)])
    pltpu.sync_copy(rsum_hbm, rbuf)
    pltpu.sync_copy(rsumsq_hbm, qbuf)
    pltpu.sync_copy(wrow_hbm, wbuf)
    pltpu.sync_copy(brow_hbm, bbuf)
    pltpu.sync_copy(b_hbm, bvec_v)

    bregs = [bvec_v[pl.ds(16 * j, 16)] for j in range(_GJ)]
    gvecs = [iota + 16 * j for j in range(_GJ)]
    lane0 = iota == 0
    zero16 = jnp.zeros((16,), jnp.int32)
    for j in range(4):
        sbuf[pl.ds(16 * j, 16)] = zero16

    cnt = [jnp.zeros((16,), jnp.float32) for _ in range(_GJ)]
    gsum = [jnp.zeros((16,), jnp.float32) for _ in range(_GJ)]
    gsumsq = [jnp.zeros((16,), jnp.float32) for _ in range(_GJ)]

    for n in range(nb):
        rv0 = plsc.load_gather(rbuf, [iota + (n * nc + c0)])
        rv1 = plsc.load_gather(rbuf, [iota + (n * nc + c0 + 8)])
        qv0 = plsc.load_gather(qbuf, [iota + (n * nc + c0)])
        qv1 = plsc.load_gather(qbuf, [iota + (n * nc + c0 + 8)])
        for i in range(cw):
            base = (n * cw + i) * _GROUP
            vals = [
                lbuf[pl.ds(base + 16 * j, 16)] + bregs[j]
                for j in range(_GJ)
            ]
            m = jnp.max(vals[0])
            for j in range(1, _GJ):
                m = jnp.maximum(m, jnp.max(vals[j]))
            s = jnp.full((16,), _GROUP, jnp.int32)
            for j in range(_GJ - 1, -1, -1):
                mask = vals[j] == m
                hit = plsc.all_reduce_population_count(mask) > 0
                idx = plsc.all_reduce_ffs(mask) + 16 * j
                s = jnp.where(hit, idx, s)
            rv = rv0[i] if i < 16 else rv1[i - 8]
            qv = qv0[i] if i < 16 else qv1[i - 8]
            for j in range(_GJ):
                eq = gvecs[j] == s
                cnt[j] = cnt[j] + jnp.where(eq, 1.0, 0.0)
                gsum[j] = gsum[j] + jnp.where(eq, rv, 0.0)
                gsumsq[j] = gsumsq[j] + jnp.where(eq, qv, 0.0)
            plsc.store_scatter(sbuf, [jnp.full((16,), n * cw + i, jnp.int32)],
                               s, mask=lane0)

    for j in range(_GJ):
        stage[pl.ds(16 * j, 16)] = cnt[j]
        stage[pl.ds(64 + 16 * j, 16)] = gsum[j]
        stage[pl.ds(128 + 16 * j, 16)] = gsumsq[j]
    pltpu.sync_copy(stage, shared.at[sid])
    plsc.subcore_barrier()
    pltpu.sync_copy(shared, allbuf)

    for j in range(_GJ):
        cnt_g = jnp.zeros((16,), jnp.float32)
        sum_g = jnp.zeros((16,), jnp.float32)
        sumsq_g = jnp.zeros((16,), jnp.float32)
        for t in range(16):
            cnt_g = cnt_g + allbuf[t, pl.ds(16 * j, 16)]
            sum_g = sum_g + allbuf[t, pl.ds(64 + 16 * j, 16)]
            sumsq_g = sumsq_g + allbuf[t, pl.ds(128 + 16 * j, 16)]
        n_el = cnt_g * d
        mean_g = sum_g / jnp.maximum(n_el, 1.0)
        var_g = (sumsq_g - n_el * mean_g * mean_g) / jnp.maximum(
            n_el - 1.0, 1.0)
        inv_g = _rsqrt_newton(var_g + _EPS)
        meanbuf[pl.ds(16 * j, 16)] = mean_g
        invbuf[pl.ds(16 * j, 16)] = inv_g

    hi = nb * nc - 1
    for k in range(2):
        svec = plsc.load_gather(sbuf, [iota + (cid * cw + 16 * k)])
        mvals = plsc.load_gather(meanbuf, [svec])
        ivals = plsc.load_gather(invbuf, [svec])
        widx = jnp.minimum(iota + (cid * nc + c0 + 16 * k), hi)
        wvals = plsc.load_gather(wbuf, [widx])
        bvals = plsc.load_gather(bbuf, [widx])
        sc = wvals * ivals
        outs[pl.ds(16 * k, 16)] = sc
        outsh[pl.ds(16 * k, 16)] = bvals - mvals * sc
    off = pl.multiple_of((cid * 16 + sid) * 32, 32)
    pltpu.sync_copy(outs, scale_hbm.at[pl.ds(off, 32)])
    pltpu.sync_copy(outsh, shift_hbm.at[pl.ds(off, 32)])


def kernel(x, fc_w, fc_b, weight, bias):
    n, c, h, w = x.shape
    d = h * w
    xm = x.transpose(0, 2, 3, 1).reshape(n, d, c)
    wrow = jnp.broadcast_to(weight.reshape(1, c), (n, c))
    brow = jnp.broadcast_to(bias.reshape(1, c), (n, c))

    kblk = _kblk(d)
    ksteps = d // kblk

    logits, rsum, rsumsq = pl.pallas_call(
        functools.partial(_pass1_body, nb=n),
        grid=(ksteps,),
        in_specs=[
            pl.BlockSpec((n, kblk, c), lambda k: (0, k, 0)),
            pl.BlockSpec((_GROUP, kblk), lambda k: (0, k)),
        ],
        out_specs=[
            pl.BlockSpec((n, _GROUP, c), lambda k: (0, 0, 0)),
            pl.BlockSpec((n, c), lambda k: (0, 0)),
            pl.BlockSpec((n, c), lambda k: (0, 0)),
        ],
        out_shape=[
            jax.ShapeDtypeStruct((n, _GROUP, c), jnp.float32),
            jax.ShapeDtypeStruct((n, c), jnp.float32),
            jax.ShapeDtypeStruct((n, c), jnp.float32),
        ],
        compiler_params=pltpu.CompilerParams(
            dimension_semantics=("arbitrary",),
            vmem_limit_bytes=60 * 1024 * 1024),
    )(xm, fc_w)

    mesh = plsc.VectorSubcoreMesh(core_axis_name="c", subcore_axis_name="s")
    cp = pltpu.CompilerParams()
    if "needs_layout_passes" in pltpu.CompilerParams.__dataclass_fields__:
        cp = dataclasses.replace(cp, needs_layout_passes=False)
    cw = c // 16
    sc_stats = pl.kernel(
        functools.partial(_sc_stats_body, nb=n, nc=c, d=float(d)),
        out_type=[
            jax.ShapeDtypeStruct((n * 16 * 32,), jnp.float32),
            jax.ShapeDtypeStruct((n * 16 * 32,), jnp.float32),
        ],
        mesh=mesh,
        scratch_types=[
            pltpu.VMEM((n * (c // 16) * _GROUP,), jnp.float32),
            pltpu.VMEM((n * c,), jnp.float32),
            pltpu.VMEM((n * c,), jnp.float32),
            pltpu.VMEM((n * c,), jnp.float32),
            pltpu.VMEM((n * c,), jnp.float32),
            pltpu.VMEM((_GROUP,), jnp.float32),
            pltpu.VMEM((256,), jnp.float32),
            pltpu.VMEM_SHARED((16, 256), jnp.float32),
            pltpu.VMEM((16, 256), jnp.float32),
            pltpu.VMEM((64,), jnp.int32),
            pltpu.VMEM((_GROUP,), jnp.float32),
            pltpu.VMEM((_GROUP,), jnp.float32),
            pltpu.VMEM((32,), jnp.float32),
            pltpu.VMEM((32,), jnp.float32),
        ],
        compiler_params=cp,
    )
    logits_t = logits.transpose(0, 2, 1).reshape(-1)
    scale, shift = sc_stats(logits_t, fc_b,
                            rsum.reshape(-1), rsumsq.reshape(-1),
                            wrow.reshape(-1), brow.reshape(-1))
    scale = scale.reshape(n, 16, 32)[:, :, :cw].reshape(n, c)
    shift = shift.reshape(n, 16, 32)[:, :, :cw].reshape(n, c)

    out = pl.pallas_call(
        _pass3_body,
        grid=(ksteps,),
        in_specs=[
            pl.BlockSpec((n, kblk, c), lambda k: (0, k, 0)),
            pl.BlockSpec((n, c), lambda k: (0, 0)),
            pl.BlockSpec((n, c), lambda k: (0, 0)),
        ],
        out_specs=pl.BlockSpec((n, kblk, c), lambda k: (0, k, 0)),
        out_shape=jax.ShapeDtypeStruct((n, d, c), jnp.float32),
        compiler_params=pltpu.CompilerParams(
            dimension_semantics=("arbitrary",),
            vmem_limit_bytes=60 * 1024 * 1024),
    )(xm, scale, shift)

    return out.reshape(n, h, w, c).transpose(0, 3, 1, 2)
